# baseline (speedup 1.0000x reference)
import functools

import jax
import jax.numpy as jnp
from jax import lax
from jax.experimental import pallas as pl
from jax.experimental.pallas import tpu as pltpu
from jax.experimental.pallas import tpu_sc as plsc

VOCAB = 100000
EMBED_DIM = 16

NC = 2
NSC = 16

IDX_PER_DMA = 128
K = 5
T = K * IDX_PER_DMA
S = 80
ROWS_PER_TEC_LOAD = VOCAB // NSC


def _sc_dual_gather(tables, idx):
  n = NSC * S * T
  mesh = plsc.VectorSubcoreMesh(core_axis_name="c", subcore_axis_name="s")

  @functools.partial(
      pl.kernel,
      out_type=jax.ShapeDtypeStruct((n, 2 * EMBED_DIM), jnp.float32),
      mesh=mesh,
      scratch_types=[
          pltpu.VMEM_SHARED((VOCAB, EMBED_DIM), jnp.float32),
          pltpu.VMEM((K, IDX_PER_DMA), jnp.int32),
          pltpu.VMEM((T, EMBED_DIM), jnp.float32),
          pltpu.SemaphoreType.DMA,
      ],
      compiler_params=pltpu.CompilerParams(use_tc_tiling_on_sc=False),
  )
  def k(tables_hbm, idx_hbm, out_hbm, table_sh, idx_v, rows_v, sem):
    cid = lax.axis_index("c")
    tid = lax.axis_index("s")

    slab = tid * ROWS_PER_TEC_LOAD
    pltpu.sync_copy(
        tables_hbm.at[pl.ds(cid * VOCAB + slab, ROWS_PER_TEC_LOAD)],
        table_sh.at[pl.ds(slab, ROWS_PER_TEC_LOAD)],
    )
    plsc.subcore_barrier()

    def step(s, carry):
      blk = tid * S + s
      pltpu.sync_copy(idx_hbm.at[cid, blk], idx_v)
      cps = [
          pltpu.async_copy(
              table_sh.at[idx_v.at[j]],
              rows_v.at[pl.ds(j * IDX_PER_DMA, IDX_PER_DMA)],
              sem,
          )
          for j in range(K)
      ]
      for c in cps:
        c.wait()
      pltpu.sync_copy(
          rows_v,
          out_hbm.at[pl.ds(blk * T, T), pl.ds(cid * EMBED_DIM, EMBED_DIM)],
      )
      return carry

    lax.fori_loop(0, S, step, 0)

  return k(tables, idx)


def kernel(prefix_indices, suffix_indices, prefix_table, suffix_table):
  batch, hist = prefix_indices.shape
  pidx = prefix_indices.reshape(-1).astype(jnp.int32)
  sidx = suffix_indices.reshape(-1).astype(jnp.int32)
  comb = jnp.stack([pidx, sidx]).reshape(NC, NSC * S, K, IDX_PER_DMA)
  cat = jnp.concatenate([prefix_table, suffix_table], axis=0)
  out = _sc_dual_gather(cat, comb)
  return out.reshape(batch, hist, 2 * EMBED_DIM)


# speedup vs baseline: 1.0646x; 1.0646x over previous
import functools

import jax
import jax.numpy as jnp
from jax import lax
from jax.experimental import pallas as pl
from jax.experimental.pallas import tpu as pltpu
from jax.experimental.pallas import tpu_sc as plsc

VOCAB = 100000
EMBED_DIM = 16

NC = 2
NSC = 16

IDX_PER_DMA = 128
K = 5
T = K * IDX_PER_DMA
S = 80
ROWS_PER_TEC_LOAD = VOCAB // NSC


def _sc_dual_gather(tables, idx):
  n = NSC * S * T
  mesh = plsc.VectorSubcoreMesh(core_axis_name="c", subcore_axis_name="s")

  @functools.partial(
      pl.kernel,
      out_type=jax.ShapeDtypeStruct((n, 2 * EMBED_DIM), jnp.float32),
      mesh=mesh,
      scratch_types=[
          pltpu.VMEM_SHARED((VOCAB, EMBED_DIM), jnp.float32),
          pltpu.VMEM((2, K, IDX_PER_DMA), jnp.int32),
          pltpu.VMEM((2, T, EMBED_DIM), jnp.float32),
          pltpu.SemaphoreType.DMA,
          pltpu.SemaphoreType.DMA,
          pltpu.SemaphoreType.DMA,
      ],
      compiler_params=pltpu.CompilerParams(use_tc_tiling_on_sc=False),
  )
  def k(tables_hbm, idx_hbm, out_hbm, table_sh, idx_v, rows_v,
        sem_g, sem_i, sem_w):
    cid = lax.axis_index("c")
    tid = lax.axis_index("s")

    slab = tid * ROWS_PER_TEC_LOAD
    pltpu.sync_copy(
        tables_hbm.at[pl.ds(cid * VOCAB + slab, ROWS_PER_TEC_LOAD)],
        table_sh.at[pl.ds(slab, ROWS_PER_TEC_LOAD)],
    )
    plsc.subcore_barrier()

    base = tid * S
    pltpu.async_copy(idx_hbm.at[cid, base], idx_v.at[0], sem_i)

    def wait_idx():
      pltpu.make_async_copy(idx_hbm.at[cid, base], idx_v.at[0], sem_i).wait()

    def wait_wb(blk):
      pltpu.make_async_copy(
          rows_v.at[0],
          out_hbm.at[pl.ds(blk * T, T), pl.ds(cid * EMBED_DIM, EMBED_DIM)],
          sem_w,
      ).wait()

    def gather_step(buf, blk):
      cps = [
          pltpu.async_copy(
              table_sh.at[idx_v.at[buf, j]],
              rows_v.at[buf, pl.ds(j * IDX_PER_DMA, IDX_PER_DMA)],
              sem_g,
          )
          for j in range(K)
      ]
      for c in cps:
        c.wait()
      pltpu.async_copy(
          rows_v.at[buf],
          out_hbm.at[pl.ds(blk * T, T), pl.ds(cid * EMBED_DIM, EMBED_DIM)],
          sem_w,
      )

    def body(i, carry):
      blk = base + 2 * i
      wait_idx()
      pltpu.async_copy(idx_hbm.at[cid, blk + 1], idx_v.at[1], sem_i)

      @pl.when(i >= 1)
      def _():
        wait_wb(blk)

      gather_step(0, blk)
      wait_idx()

      @pl.when(i < S // 2 - 1)
      def _():
        pltpu.async_copy(idx_hbm.at[cid, blk + 2], idx_v.at[0], sem_i)

      @pl.when(i >= 1)
      def _():
        wait_wb(blk)

      gather_step(1, blk + 1)
      return carry

    lax.fori_loop(0, S // 2, body, 0)
    wait_wb(base + S - 2)
    wait_wb(base + S - 1)

  return k(tables, idx)


def kernel(prefix_indices, suffix_indices, prefix_table, suffix_table):
  batch, hist = prefix_indices.shape
  pidx = prefix_indices.reshape(-1).astype(jnp.int32)
  sidx = suffix_indices.reshape(-1).astype(jnp.int32)
  comb = jnp.stack([pidx, sidx]).reshape(NC, NSC * S, K, IDX_PER_DMA)
  cat = jnp.concatenate([prefix_table, suffix_table], axis=0)
  out = _sc_dual_gather(cat, comb)
  return out.reshape(batch, hist, 2 * EMBED_DIM)
